# baseline (speedup 1.0000x reference)
import functools

import jax
import jax.numpy as jnp
from jax import lax
from jax.experimental import pallas as pl
from jax.experimental.pallas import tpu as pltpu
from jax.experimental.pallas import tpu_sc as plsc

NC = 2
NS = 16
L = 16
HEAD = 4
NORMW = 16

_i32 = jnp.int32
_f32 = jnp.float32

_SC_PARAMS = pltpu.CompilerParams(
    needs_layout_passes=False, use_tc_tiling_on_sc=False)


def _iota16():
    return lax.iota(_i32, L)


def _take(v, idx):
    dnums = lax.GatherDimensionNumbers(
        offset_dims=(), collapsed_slice_dims=(0,), start_index_map=(0,))
    return lax.gather(v, idx[:, None], dnums, (1,),
                      mode=lax.GatherScatterMode.PROMISE_IN_BOUNDS)


def _fill2d(ref, nrows, ncols, val):
    vvec = jnp.full((L,), val, _f32)
    def body(i, _):
        flat = i * L + _iota16()
        plsc.store_scatter(ref, [flat // ncols, flat % ncols], vvec)
        return 0
    lax.fori_loop(0, nrows * ncols // L, body, 0)


def _qk(embeds, qT, kT):
    n, d = embeds.shape
    br = 1000
    def body(e_ref, q_ref, k_ref, oq, ok):
        x = e_ref[...]
        oq[...] = jnp.dot(x, q_ref[...], preferred_element_type=_f32)
        ok[...] = jnp.dot(x, k_ref[...], preferred_element_type=_f32)
    return pl.pallas_call(
        body,
        grid=(n // br,),
        in_specs=[pl.BlockSpec((br, d), lambda i: (i, 0)),
                  pl.BlockSpec((d, d), lambda i: (0, 0)),
                  pl.BlockSpec((d, d), lambda i: (0, 0))],
        out_specs=[pl.BlockSpec((br, d), lambda i: (i, 0))] * 2,
        out_shape=[jax.ShapeDtypeStruct((n, d), _f32)] * 2,
    )(embeds, qT, kT)


def _vnorm(embeds, vT, n0, n1):
    n, d = embeds.shape
    nw = n0.shape[1]
    br = 1000
    def body(e_ref, v_ref, n0_ref, n1_ref, ov, on):
        ov[...] = jnp.dot(e_ref[...], v_ref[...], preferred_element_type=_f32)
        on[...] = n0_ref[...] + n1_ref[...]
    return pl.pallas_call(
        body,
        grid=(n // br,),
        in_specs=[pl.BlockSpec((br, d), lambda i: (i, 0)),
                  pl.BlockSpec((d, d), lambda i: (0, 0)),
                  pl.BlockSpec((br, nw), lambda i: (i, 0)),
                  pl.BlockSpec((br, nw), lambda i: (i, 0))],
        out_specs=[pl.BlockSpec((br, d), lambda i: (i, 0)),
                   pl.BlockSpec((br, nw), lambda i: (i, 0))],
        out_shape=[jax.ShapeDtypeStruct((n, d), _f32),
                   jax.ShapeDtypeStruct((n, nw), _f32)],
    )(embeds, vT, n0, n1)


def _combine(a, b):
    n, d = a.shape
    br = 1000
    def body(a_ref, b_ref, o_ref):
        o_ref[...] = a_ref[...] + b_ref[...]
    return pl.pallas_call(
        body,
        grid=(n // br,),
        in_specs=[pl.BlockSpec((br, d), lambda i: (i, 0))] * 2,
        out_specs=pl.BlockSpec((br, d), lambda i: (i, 0)),
        out_shape=jax.ShapeDtypeStruct((n, d), _f32),
    )(a, b)


def _make_pass1(n_nodes, n_edges, dim, c_edges):
    epw = n_edges // (NC * NS)
    nchunks = epw // c_edges
    npairs = nchunks // 2
    mesh = plsc.VectorSubcoreMesh(core_axis_name="c", subcore_axis_name="s",
                                  num_cores=NC, num_subcores=NS)

    @functools.partial(
        pl.kernel,
        out_type=(jax.ShapeDtypeStruct((n_edges, HEAD), _f32),
                  jax.ShapeDtypeStruct((n_nodes, NORMW), _f32),
                  jax.ShapeDtypeStruct((n_nodes, NORMW), _f32)),
        mesh=mesh,
        compiler_params=_SC_PARAMS,
        scratch_types=[
            pltpu.VMEM((nchunks, c_edges), _i32),
            pltpu.VMEM((nchunks, c_edges), _i32),
            pltpu.VMEM((2, c_edges, dim), _f32),
            pltpu.VMEM((2, c_edges, dim), _f32),
            pltpu.VMEM((2, c_edges, HEAD), _f32),
            pltpu.VMEM((2, c_edges, NORMW), _f32),
            pltpu.VMEM((200, NORMW), _f32),
            pltpu.SemaphoreType.DMA,
            pltpu.SemaphoreType.DMA,
            pltpu.SemaphoreType.DMA,
            pltpu.SemaphoreType.DMA,
            pltpu.VMEM_SHARED((n_nodes, NORMW), _f32),
        ],
    )
    def pass1(rows3_hbm, cols3_hbm, q_hbm, k_hbm,
              expatt_hbm, norm0_hbm, norm1_hbm,
              sidx, scol, qbuf, kbuf, attc, attp, znorm,
              semg0, semg1, semw0, semw1, norm_acc):
        c = lax.axis_index("c")
        s = lax.axis_index("s")
        wid = c * NS + s
        semg = (semg0, semg1)
        semw = (semw0, semw1)
        nzt = 10
        rpt = n_nodes // nzt
        hd = dim // HEAD

        pltpu.sync_copy(rows3_hbm.at[wid], sidx)
        pltpu.sync_copy(cols3_hbm.at[wid], scol)

        for b in (0, 1):
            pltpu.async_copy(q_hbm.at[sidx.at[b]], qbuf.at[b], semg[b])
            pltpu.async_copy(k_hbm.at[scol.at[b]], kbuf.at[b], semg[b])

        _fill2d(attp.at[0], c_edges, NORMW, 0.0)
        _fill2d(attp.at[1], c_edges, NORMW, 0.0)
        _fill2d(znorm, 200, NORMW, 0.0)
        @pl.when(s < nzt)
        def _():
            def zb(i, _):
                pltpu.sync_copy(znorm,
                                norm_acc.at[pl.ds(s * rpt + i * 200, 200), :])
                return 0
            lax.fori_loop(0, rpt // 200, zb, 0)
        plsc.subcore_barrier()

        ii = _iota16()
        r8 = ii ^ 8
        r4 = ii ^ 4
        r2 = ii ^ 2
        r1 = ii ^ 1
        qid = ii // HEAD
        m0 = qid == 0
        m1 = qid == 1
        m2 = qid == 2
        smask = (ii % HEAD) == 0

        def chunk_work(g, b):
            base = wid * epw + g * c_edges
            qb = qbuf.at[b]
            kb = kbuf.at[b]
            ab = attc.at[b]
            pb = attp.at[b]
            pltpu.make_async_copy(q_hbm.at[sidx.at[g]], qb, semg[b]).wait()
            pltpu.make_async_copy(k_hbm.at[scol.at[g]], kb, semg[b]).wait()
            @pl.when(g >= 2)
            def _():
                pltpu.make_async_copy(
                    ab, expatt_hbm.at[pl.ds(base, c_edges)], semw[b]).wait()

            def edge(e, _):
                ph = []
                for h in range(HEAD):
                    p = qb[e, pl.ds(h * hd, L)] * kb[e, pl.ds(h * hd, L)]
                    for j in range(1, hd // L):
                        off = h * hd + j * L
                        p = p + qb[e, pl.ds(off, L)] * kb[e, pl.ds(off, L)]
                    p = p + _take(p, r8)
                    p = p + _take(p, r4)
                    ph.append(p)
                d = jnp.where(m0, ph[0],
                              jnp.where(m1, ph[1],
                                        jnp.where(m2, ph[2], ph[3])))
                f = d + _take(d, r2)
                f = f + _take(f, r1)
                plsc.store_scatter(ab, [jnp.full((L,), e, _i32), qid],
                                   f, mask=smask)
                return 0
            lax.fori_loop(0, c_edges, edge, 0)

            def pgrp(i2, _):
                flat = i2 * L + _iota16()
                ee = flat // HEAD
                hh = flat % HEAD
                raw = plsc.load_gather(ab, [ee, hh])
                v = jnp.exp(jnp.clip(raw, -10.0, 10.0))
                plsc.store_scatter(ab, [ee, hh], v)
                plsc.store_scatter(pb, [ee, hh], v)
                return 0
            lax.fori_loop(0, c_edges * HEAD // L, pgrp, 0)

            pltpu.async_copy(ab, expatt_hbm.at[pl.ds(base, c_edges)], semw[b])
            pltpu.sync_copy(pb, norm_acc.at[sidx.at[g]], add=True)
            @pl.when(g + 2 < nchunks)
            def _():
                pltpu.async_copy(q_hbm.at[sidx.at[g + 2]], qb, semg[b])
                pltpu.async_copy(k_hbm.at[scol.at[g + 2]], kb, semg[b])

        def pair(gp, _):
            chunk_work(gp * 2, 0)
            chunk_work(gp * 2 + 1, 1)
            return 0
        lax.fori_loop(0, npairs, pair, 0)

        for b in (0, 1):
            g_last = nchunks - 2 + b
            base = wid * epw + g_last * c_edges
            pltpu.make_async_copy(
                attc.at[b], expatt_hbm.at[pl.ds(base, c_edges)],
                semw[b]).wait()

        plsc.subcore_barrier()
        @pl.when(jnp.logical_and(s < nzt, c == 0))
        def _():
            pltpu.sync_copy(norm_acc.at[pl.ds(s * rpt, rpt), :],
                            norm0_hbm.at[pl.ds(s * rpt, rpt), :])

        @pl.when(jnp.logical_and(s < nzt, c == 1))
        def _():
            pltpu.sync_copy(norm_acc.at[pl.ds(s * rpt, rpt), :],
                            norm1_hbm.at[pl.ds(s * rpt, rpt), :])

    return pass1


def _make_pass2(n_nodes, n_edges, dim, c_edges):
    epw = n_edges // (NC * NS)
    nchunks = epw // c_edges
    npairs = nchunks // 2
    mesh = plsc.VectorSubcoreMesh(core_axis_name="c", subcore_axis_name="s",
                                  num_cores=NC, num_subcores=NS)

    @functools.partial(
        pl.kernel,
        out_type=(jax.ShapeDtypeStruct((n_edges, HEAD), _f32),
                  jax.ShapeDtypeStruct((NC, n_nodes, dim), _f32)),
        mesh=mesh,
        compiler_params=_SC_PARAMS,
        scratch_types=[
            pltpu.VMEM((nchunks, c_edges), _i32),
            pltpu.VMEM((2, c_edges), _i32),
            pltpu.VMEM((2, c_edges, dim), _f32),
            pltpu.VMEM((2, c_edges, HEAD), _f32),
            pltpu.VMEM((2, c_edges, HEAD), _f32),
            pltpu.VMEM((c_edges * HEAD,), _f32),
            pltpu.VMEM((2, c_edges, NORMW), _f32),
            pltpu.SemaphoreType.DMA,
            pltpu.SemaphoreType.DMA,
            pltpu.SemaphoreType.DMA,
            pltpu.SemaphoreType.DMA,
            pltpu.SemaphoreType.DMA,
            pltpu.SemaphoreType.DMA,
            pltpu.VMEM_SHARED((n_nodes, dim), _f32),
        ],
    )
    def pass2(rows3_hbm, cols3_hbm, v_hbm, expatt_hbm, norm_hbm,
              attout_hbm, accpart_hbm,
              sidx, cidx, vbuf, eabuf, attc, attf, nbuf,
              semg0, semg1, semi0, semi1, semw0, semw1, acc):
        c = lax.axis_index("c")
        s = lax.axis_index("s")
        wid = c * NS + s
        semg = (semg0, semg1)
        semi = (semi0, semi1)
        semw = (semw0, semw1)
        nzt = 10
        rpt = n_nodes // nzt
        hd = dim // HEAD
        epg = L // HEAD

        pltpu.sync_copy(rows3_hbm.at[wid], sidx)

        _fill2d(vbuf.at[0], 40, dim, 0.0)
        @pl.when(s < nzt)
        def _():
            def zb(i, _):
                pltpu.sync_copy(vbuf.at[0, pl.ds(0, 40), :],
                                acc.at[pl.ds(s * rpt + i * 40, 40), :])
                return 0
            lax.fori_loop(0, rpt // 40, zb, 0)
        plsc.subcore_barrier()

        for b in (0, 1):
            base_b = wid * epw + b * c_edges
            pltpu.sync_copy(cols3_hbm.at[wid, b], cidx.at[b])
            pltpu.async_copy(v_hbm.at[cidx.at[b]], vbuf.at[b], semg[b])
            pltpu.async_copy(norm_hbm.at[sidx.at[b]], nbuf.at[b], semg[b])
            pltpu.async_copy(expatt_hbm.at[pl.ds(base_b, c_edges)],
                             eabuf.at[b], semg[b])

        def chunk_work(g, b):
            base = wid * epw + g * c_edges
            vb = vbuf.at[b]
            eb = eabuf.at[b]
            ab = attc.at[b]
            nb = nbuf.at[b]
            @pl.when(g + 2 < nchunks)
            def _():
                pltpu.async_copy(cols3_hbm.at[wid, g + 2], cidx.at[b],
                                 semi[b])
            pltpu.make_async_copy(v_hbm.at[cidx.at[b]], vb, semg[b]).wait()
            pltpu.make_async_copy(norm_hbm.at[sidx.at[g]], nb,
                                  semg[b]).wait()
            pltpu.make_async_copy(expatt_hbm.at[pl.ds(base, c_edges)], eb,
                                  semg[b]).wait()
            @pl.when(g >= 2)
            def _():
                pltpu.make_async_copy(
                    ab, attout_hbm.at[pl.ds(base, c_edges)], semw[b]).wait()

            def pgrp(i2, _):
                flat = i2 * L + _iota16()
                ee = flat // HEAD
                hh = flat % HEAD
                ea = plsc.load_gather(eb, [ee, hh])
                nv = plsc.load_gather(nb, [ee, hh])
                att = ea / (nv + 1e-8)
                plsc.store_scatter(ab, [ee, hh], att)
                attf[pl.ds(i2 * L, L)] = att
                return 0
            lax.fori_loop(0, c_edges * HEAD // L, pgrp, 0)

            def edge4(q4, _):
                va = attf[pl.ds(q4 * L, L)]
                for eo in range(epg):
                    e = q4 * epg + eo
                    for h in range(HEAD):
                        a_h = va[eo * HEAD + h]
                        for j in range(hd // L):
                            off = h * hd + j * L
                            vb[e, pl.ds(off, L)] = vb[e, pl.ds(off, L)] * a_h
                return 0
            lax.fori_loop(0, c_edges // epg, edge4, 0)

            pltpu.async_copy(ab, attout_hbm.at[pl.ds(base, c_edges)], semw[b])
            pltpu.sync_copy(vb, acc.at[sidx.at[g]], add=True)
            @pl.when(g + 2 < nchunks)
            def _():
                base2 = wid * epw + (g + 2) * c_edges
                pltpu.make_async_copy(cols3_hbm.at[wid, g + 2], cidx.at[b],
                                      semi[b]).wait()
                pltpu.async_copy(v_hbm.at[cidx.at[b]], vb, semg[b])
                pltpu.async_copy(norm_hbm.at[sidx.at[g + 2]], nb, semg[b])
                pltpu.async_copy(expatt_hbm.at[pl.ds(base2, c_edges)],
                                 eabuf.at[b], semg[b])

        def pair(gp, _):
            chunk_work(gp * 2, 0)
            chunk_work(gp * 2 + 1, 1)
            return 0
        lax.fori_loop(0, npairs, pair, 0)

        for b in (0, 1):
            g_last = nchunks - 2 + b
            base = wid * epw + g_last * c_edges
            pltpu.make_async_copy(
                attc.at[b], attout_hbm.at[pl.ds(base, c_edges)],
                semw[b]).wait()

        plsc.subcore_barrier()
        @pl.when(s < nzt)
        def _():
            pltpu.sync_copy(
                acc.at[pl.ds(s * rpt, rpt), :],
                accpart_hbm.at[c, pl.ds(s * rpt, rpt), :])

    return pass2


def kernel(adj, embeds, qTrans, kTrans, vTrans):
    n_nodes, dim = embeds.shape
    n_edges = adj.shape[1]
    c_edges = 40
    nw = NC * NS
    nchunks = n_edges // (nw * c_edges)
    rows3 = adj[0].reshape(nw, nchunks, c_edges)
    cols3 = adj[1].reshape(nw, nchunks, c_edges)

    q, k = _qk(embeds, qTrans, kTrans)

    expatt, norm0, norm1 = _make_pass1(n_nodes, n_edges, dim, c_edges)(
        rows3, cols3, q, k)
    v, nsum = _vnorm(embeds, vTrans, norm0, norm1)
    att, accpart = _make_pass2(n_nodes, n_edges, dim, c_edges)(
        rows3, cols3, v, expatt, nsum)
    res = _combine(accpart[0], accpart[1])
    return res, att


# speedup vs baseline: 1.0363x; 1.0363x over previous
import functools

import jax
import jax.numpy as jnp
from jax import lax
from jax.experimental import pallas as pl
from jax.experimental.pallas import tpu as pltpu
from jax.experimental.pallas import tpu_sc as plsc

NC = 2
NS = 16
L = 16
HEAD = 4
NORMW = 16

_i32 = jnp.int32
_f32 = jnp.float32

_SC_PARAMS = pltpu.CompilerParams(
    needs_layout_passes=False, use_tc_tiling_on_sc=False)


def _iota16():
    return lax.iota(_i32, L)


def _take(v, idx):
    dnums = lax.GatherDimensionNumbers(
        offset_dims=(), collapsed_slice_dims=(0,), start_index_map=(0,))
    return lax.gather(v, idx[:, None], dnums, (1,),
                      mode=lax.GatherScatterMode.PROMISE_IN_BOUNDS)


def _fill2d(ref, nrows, ncols, val):
    vvec = jnp.full((L,), val, _f32)
    def body(i, _):
        flat = i * L + _iota16()
        plsc.store_scatter(ref, [flat // ncols, flat % ncols], vvec)
        return 0
    lax.fori_loop(0, nrows * ncols // L, body, 0)


def _qk(embeds, qT, kT):
    n, d = embeds.shape
    br = 1000
    def body(e_ref, q_ref, k_ref, oq, ok):
        x = e_ref[...]
        oq[...] = jnp.dot(x, q_ref[...], preferred_element_type=_f32)
        ok[...] = jnp.dot(x, k_ref[...], preferred_element_type=_f32)
    return pl.pallas_call(
        body,
        grid=(n // br,),
        in_specs=[pl.BlockSpec((br, d), lambda i: (i, 0)),
                  pl.BlockSpec((d, d), lambda i: (0, 0)),
                  pl.BlockSpec((d, d), lambda i: (0, 0))],
        out_specs=[pl.BlockSpec((br, d), lambda i: (i, 0))] * 2,
        out_shape=[jax.ShapeDtypeStruct((n, d), _f32)] * 2,
    )(embeds, qT, kT)


def _vmat(embeds, vT):
    n, d = embeds.shape
    br = 1000
    def body(e_ref, v_ref, ov):
        ov[...] = jnp.dot(e_ref[...], v_ref[...], preferred_element_type=_f32)
    return pl.pallas_call(
        body,
        grid=(n // br,),
        in_specs=[pl.BlockSpec((br, d), lambda i: (i, 0)),
                  pl.BlockSpec((d, d), lambda i: (0, 0))],
        out_specs=pl.BlockSpec((br, d), lambda i: (i, 0)),
        out_shape=jax.ShapeDtypeStruct((n, d), _f32),
    )(embeds, vT)


def _combine(a, b):
    n, d = a.shape
    br = 1000
    def body(a_ref, b_ref, o_ref):
        o_ref[...] = a_ref[...] + b_ref[...]
    return pl.pallas_call(
        body,
        grid=(n // br,),
        in_specs=[pl.BlockSpec((br, d), lambda i: (i, 0))] * 2,
        out_specs=pl.BlockSpec((br, d), lambda i: (i, 0)),
        out_shape=jax.ShapeDtypeStruct((n, d), _f32),
    )(a, b)


def _make_pass1(n_nodes, n_edges, dim, c_edges):
    epw = n_edges // (NC * NS)
    nchunks = epw // c_edges
    npairs = nchunks // 2
    mesh = plsc.VectorSubcoreMesh(core_axis_name="c", subcore_axis_name="s",
                                  num_cores=NC, num_subcores=NS)

    @functools.partial(
        pl.kernel,
        out_type=(jax.ShapeDtypeStruct((n_edges, HEAD), _f32),
                  jax.ShapeDtypeStruct((n_nodes, NORMW), _f32),
                  jax.ShapeDtypeStruct((n_nodes, NORMW), _f32)),
        mesh=mesh,
        compiler_params=_SC_PARAMS,
        scratch_types=[
            pltpu.VMEM((nchunks, c_edges), _i32),
            pltpu.VMEM((2, c_edges), _i32),
            pltpu.VMEM((2, c_edges, dim), _f32),
            pltpu.VMEM((2, c_edges, dim), _f32),
            pltpu.VMEM((2, c_edges, HEAD), _f32),
            pltpu.VMEM((2, c_edges, NORMW), _f32),
            pltpu.VMEM((200, NORMW), _f32),
            pltpu.SemaphoreType.DMA,
            pltpu.SemaphoreType.DMA,
            pltpu.SemaphoreType.DMA,
            pltpu.SemaphoreType.DMA,
            pltpu.SemaphoreType.DMA,
            pltpu.SemaphoreType.DMA,
            pltpu.VMEM_SHARED((n_nodes, NORMW), _f32),
        ],
    )
    def pass1(rows3_hbm, cols3_hbm, q_hbm, k_hbm,
              expatt_hbm, norm0_hbm, norm1_hbm,
              sidx, scol, qbuf, kbuf, attc, attp, znorm,
              semg0, semg1, semw0, semw1, semi0, semi1, norm_acc):
        c = lax.axis_index("c")
        s = lax.axis_index("s")
        wid = c * NS + s
        semg = (semg0, semg1)
        semw = (semw0, semw1)
        semi = (semi0, semi1)
        nzt = 10
        rpt = n_nodes // nzt
        hd = dim // HEAD

        pltpu.sync_copy(rows3_hbm.at[wid], sidx)

        for b in (0, 1):
            pltpu.sync_copy(cols3_hbm.at[wid, b], scol.at[b])
            pltpu.async_copy(q_hbm.at[sidx.at[b]], qbuf.at[b], semg[b])
            pltpu.async_copy(k_hbm.at[scol.at[b]], kbuf.at[b], semg[b])

        _fill2d(attp.at[0], c_edges, NORMW, 0.0)
        _fill2d(attp.at[1], c_edges, NORMW, 0.0)
        _fill2d(znorm, 200, NORMW, 0.0)
        @pl.when(s < nzt)
        def _():
            def zb(i, _):
                pltpu.sync_copy(znorm,
                                norm_acc.at[pl.ds(s * rpt + i * 200, 200), :])
                return 0
            lax.fori_loop(0, rpt // 200, zb, 0)
        plsc.subcore_barrier()

        ii = _iota16()
        r8 = ii ^ 8
        r4 = ii ^ 4
        r2 = ii ^ 2
        r1 = ii ^ 1
        qid = ii // HEAD
        m0 = qid == 0
        m1 = qid == 1
        m2 = qid == 2
        smask = (ii % HEAD) == 0

        def chunk_work(g, b):
            base = wid * epw + g * c_edges
            qb = qbuf.at[b]
            kb = kbuf.at[b]
            ab = attc.at[b]
            pb = attp.at[b]
            pltpu.make_async_copy(q_hbm.at[sidx.at[g]], qb, semg[b]).wait()
            pltpu.make_async_copy(k_hbm.at[scol.at[b]], kb, semg[b]).wait()
            @pl.when(g + 2 < nchunks)
            def _():
                pltpu.async_copy(cols3_hbm.at[wid, g + 2], scol.at[b],
                                 semi[b])
            @pl.when(g >= 2)
            def _():
                pltpu.make_async_copy(
                    ab, expatt_hbm.at[pl.ds(base, c_edges)], semw[b]).wait()

            def edge(e, _):
                ph = []
                for h in range(HEAD):
                    p = qb[e, pl.ds(h * hd, L)] * kb[e, pl.ds(h * hd, L)]
                    for j in range(1, hd // L):
                        off = h * hd + j * L
                        p = p + qb[e, pl.ds(off, L)] * kb[e, pl.ds(off, L)]
                    p = p + _take(p, r8)
                    p = p + _take(p, r4)
                    ph.append(p)
                d = jnp.where(m0, ph[0],
                              jnp.where(m1, ph[1],
                                        jnp.where(m2, ph[2], ph[3])))
                f = d + _take(d, r2)
                f = f + _take(f, r1)
                plsc.store_scatter(ab, [jnp.full((L,), e, _i32), qid],
                                   f, mask=smask)
                return 0
            lax.fori_loop(0, c_edges, edge, 0)

            def pgrp(i2, _):
                flat = i2 * L + _iota16()
                ee = flat // HEAD
                hh = flat % HEAD
                raw = plsc.load_gather(ab, [ee, hh])
                v = jnp.exp(jnp.clip(raw, -10.0, 10.0))
                plsc.store_scatter(ab, [ee, hh], v)
                plsc.store_scatter(pb, [ee, hh], v)
                return 0
            lax.fori_loop(0, c_edges * HEAD // L, pgrp, 0)

            pltpu.async_copy(ab, expatt_hbm.at[pl.ds(base, c_edges)], semw[b])
            pltpu.sync_copy(pb, norm_acc.at[sidx.at[g]], add=True)
            @pl.when(g + 2 < nchunks)
            def _():
                pltpu.make_async_copy(cols3_hbm.at[wid, g + 2], scol.at[b],
                                      semi[b]).wait()
                pltpu.async_copy(q_hbm.at[sidx.at[g + 2]], qb, semg[b])
                pltpu.async_copy(k_hbm.at[scol.at[b]], kb, semg[b])

        def pair(gp, _):
            chunk_work(gp * 2, 0)
            chunk_work(gp * 2 + 1, 1)
            return 0
        lax.fori_loop(0, npairs, pair, 0)
        if nchunks % 2:
            chunk_work(jnp.int32(nchunks - 1), 0)

        for b in (0, 1):
            g_last = nchunks - 2 + b
            base = wid * epw + g_last * c_edges
            pltpu.make_async_copy(
                attc.at[b], expatt_hbm.at[pl.ds(base, c_edges)],
                semw[b]).wait()

        plsc.subcore_barrier()
        @pl.when(jnp.logical_and(s < nzt, c == 0))
        def _():
            pltpu.sync_copy(norm_acc.at[pl.ds(s * rpt, rpt), :],
                            norm0_hbm.at[pl.ds(s * rpt, rpt), :])

        @pl.when(jnp.logical_and(s < nzt, c == 1))
        def _():
            pltpu.sync_copy(norm_acc.at[pl.ds(s * rpt, rpt), :],
                            norm1_hbm.at[pl.ds(s * rpt, rpt), :])

    return pass1


def _make_pass2(n_nodes, n_edges, dim, c_edges):
    epw = n_edges // (NC * NS)
    nchunks = epw // c_edges
    npairs = nchunks // 2
    mesh = plsc.VectorSubcoreMesh(core_axis_name="c", subcore_axis_name="s",
                                  num_cores=NC, num_subcores=NS)

    @functools.partial(
        pl.kernel,
        out_type=(jax.ShapeDtypeStruct((n_edges, HEAD), _f32),
                  jax.ShapeDtypeStruct((NC, n_nodes, dim), _f32)),
        mesh=mesh,
        compiler_params=_SC_PARAMS,
        scratch_types=[
            pltpu.VMEM((nchunks, c_edges), _i32),
            pltpu.VMEM((2, c_edges), _i32),
            pltpu.VMEM((2, c_edges, dim), _f32),
            pltpu.VMEM((2, c_edges, HEAD), _f32),
            pltpu.VMEM((2, c_edges, HEAD), _f32),
            pltpu.VMEM((c_edges * HEAD,), _f32),
            pltpu.VMEM((2, c_edges, NORMW), _f32),
            pltpu.VMEM((2, c_edges, NORMW), _f32),
            pltpu.SemaphoreType.DMA,
            pltpu.SemaphoreType.DMA,
            pltpu.SemaphoreType.DMA,
            pltpu.SemaphoreType.DMA,
            pltpu.SemaphoreType.DMA,
            pltpu.SemaphoreType.DMA,
            pltpu.VMEM_SHARED((n_nodes, dim), _f32),
        ],
    )
    def pass2(rows3_hbm, cols3_hbm, v_hbm, expatt_hbm, norm0_hbm, norm1_hbm,
              attout_hbm, accpart_hbm,
              sidx, cidx, vbuf, eabuf, attc, attf, nbuf0, nbuf1,
              semg0, semg1, semi0, semi1, semw0, semw1, acc):
        c = lax.axis_index("c")
        s = lax.axis_index("s")
        wid = c * NS + s
        semg = (semg0, semg1)
        semi = (semi0, semi1)
        semw = (semw0, semw1)
        nzt = 10
        rpt = n_nodes // nzt
        hd = dim // HEAD
        epg = L // HEAD

        pltpu.sync_copy(rows3_hbm.at[wid], sidx)

        _fill2d(vbuf.at[0], 40, dim, 0.0)
        @pl.when(s < nzt)
        def _():
            def zb(i, _):
                pltpu.sync_copy(vbuf.at[0, pl.ds(0, 40), :],
                                acc.at[pl.ds(s * rpt + i * 40, 40), :])
                return 0
            lax.fori_loop(0, rpt // 40, zb, 0)
        plsc.subcore_barrier()

        for b in (0, 1):
            base_b = wid * epw + b * c_edges
            pltpu.sync_copy(cols3_hbm.at[wid, b], cidx.at[b])
            pltpu.async_copy(v_hbm.at[cidx.at[b]], vbuf.at[b], semg[b])
            pltpu.async_copy(norm0_hbm.at[sidx.at[b]], nbuf0.at[b], semg[b])
            pltpu.async_copy(norm1_hbm.at[sidx.at[b]], nbuf1.at[b], semg[b])
            pltpu.async_copy(expatt_hbm.at[pl.ds(base_b, c_edges)],
                             eabuf.at[b], semg[b])

        def chunk_work(g, b):
            base = wid * epw + g * c_edges
            vb = vbuf.at[b]
            eb = eabuf.at[b]
            ab = attc.at[b]
            n0b = nbuf0.at[b]
            n1b = nbuf1.at[b]
            pltpu.make_async_copy(v_hbm.at[cidx.at[b]], vb, semg[b]).wait()
            @pl.when(g + 2 < nchunks)
            def _():
                pltpu.async_copy(cols3_hbm.at[wid, g + 2], cidx.at[b],
                                 semi[b])
            pltpu.make_async_copy(norm0_hbm.at[sidx.at[g]], n0b,
                                  semg[b]).wait()
            pltpu.make_async_copy(norm1_hbm.at[sidx.at[g]], n1b,
                                  semg[b]).wait()
            pltpu.make_async_copy(expatt_hbm.at[pl.ds(base, c_edges)], eb,
                                  semg[b]).wait()
            @pl.when(g >= 2)
            def _():
                pltpu.make_async_copy(
                    ab, attout_hbm.at[pl.ds(base, c_edges)], semw[b]).wait()

            def pgrp(i2, _):
                flat = i2 * L + _iota16()
                ee = flat // HEAD
                hh = flat % HEAD
                ea = plsc.load_gather(eb, [ee, hh])
                n0 = plsc.load_gather(n0b, [ee, hh])
                n1 = plsc.load_gather(n1b, [ee, hh])
                att = ea / (n0 + n1 + 1e-8)
                plsc.store_scatter(ab, [ee, hh], att)
                attf[pl.ds(i2 * L, L)] = att
                return 0
            lax.fori_loop(0, c_edges * HEAD // L, pgrp, 0)

            def edge4(q4, _):
                va = attf[pl.ds(q4 * L, L)]
                for eo in range(epg):
                    e = q4 * epg + eo
                    for h in range(HEAD):
                        a_h = va[eo * HEAD + h]
                        for j in range(hd // L):
                            off = h * hd + j * L
                            vb[e, pl.ds(off, L)] = vb[e, pl.ds(off, L)] * a_h
                return 0
            lax.fori_loop(0, c_edges // epg, edge4, 0)

            pltpu.async_copy(ab, attout_hbm.at[pl.ds(base, c_edges)], semw[b])
            pltpu.sync_copy(vb, acc.at[sidx.at[g]], add=True)
            @pl.when(g + 2 < nchunks)
            def _():
                base2 = wid * epw + (g + 2) * c_edges
                pltpu.make_async_copy(cols3_hbm.at[wid, g + 2], cidx.at[b],
                                      semi[b]).wait()
                pltpu.async_copy(v_hbm.at[cidx.at[b]], vb, semg[b])
                pltpu.async_copy(norm0_hbm.at[sidx.at[g + 2]], n0b, semg[b])
                pltpu.async_copy(norm1_hbm.at[sidx.at[g + 2]], n1b, semg[b])
                pltpu.async_copy(expatt_hbm.at[pl.ds(base2, c_edges)],
                                 eabuf.at[b], semg[b])

        def pair(gp, _):
            chunk_work(gp * 2, 0)
            chunk_work(gp * 2 + 1, 1)
            return 0
        lax.fori_loop(0, npairs, pair, 0)

        for b in (0, 1):
            g_last = nchunks - 2 + b
            base = wid * epw + g_last * c_edges
            pltpu.make_async_copy(
                attc.at[b], attout_hbm.at[pl.ds(base, c_edges)],
                semw[b]).wait()

        plsc.subcore_barrier()
        @pl.when(s < nzt)
        def _():
            pltpu.sync_copy(
                acc.at[pl.ds(s * rpt, rpt), :],
                accpart_hbm.at[c, pl.ds(s * rpt, rpt), :])

    return pass2


def kernel(adj, embeds, qTrans, kTrans, vTrans):
    n_nodes, dim = embeds.shape
    n_edges = adj.shape[1]
    nw = NC * NS
    c1 = 80
    c2 = 40
    rows3a = adj[0].reshape(nw, n_edges // (nw * c1), c1)
    cols3a = adj[1].reshape(nw, n_edges // (nw * c1), c1)
    rows3b = adj[0].reshape(nw, n_edges // (nw * c2), c2)
    cols3b = adj[1].reshape(nw, n_edges // (nw * c2), c2)

    q, k = _qk(embeds, qTrans, kTrans)
    v = _vmat(embeds, vTrans)

    expatt, norm0, norm1 = _make_pass1(n_nodes, n_edges, dim, c1)(
        rows3a, cols3a, q, k)
    att, accpart = _make_pass2(n_nodes, n_edges, dim, c2)(
        rows3b, cols3b, v, expatt, norm0, norm1)
    res = _combine(accpart[0], accpart[1])
    return res, att
